# baseline (speedup 1.0000x reference)
import functools

import numpy as np

import jax
import jax.numpy as jnp
from jax import lax
from jax.experimental import pallas as pl
from jax.experimental.pallas import tpu as pltpu
from jax.experimental.pallas import tpu_sc as plsc

N = 10000
F_IN = 128
NB = 16

NC = 2
NS = 16
L = 16
NW = NC * NS

CH = 64
FAST = 0
NCKF = 126
NCKS = 32
NCKM = NCKF
EPT_MAX = NCKM * CH
E_PAD = NS * (NCKF + NCKS) * CH

N_PAD = 10240
ROWS_PT = N_PAD // NS
WB_CHUNK = 64
WB_STEPS = ROWS_PT // WB_CHUNK

_MESH = plsc.VectorSubcoreMesh(core_axis_name="c", subcore_axis_name="s")


def _make_spmm(D, scatter_by_src=False, p_is_ones=False, p_bf16=False):
    DV = D // L
    NBUF = 2

    def body(p_h, src_h, dst_h, w_h, out_h, src_v, dst_v, w_v, *rest):
        rows = rest[:NBUF]
        fbuf = rest[NBUF] if p_bf16 else rows[0]
        acc_s = rest[NBUF + 1] if p_bf16 else rest[NBUF]
        gsem = rest[(NBUF + 2 if p_bf16 else NBUF + 1):]
        c = lax.axis_index("c")
        s = lax.axis_index("s")
        wid = s * NC + c
        cc = jnp.where(c == FAST, NCKF, NCKS)

        pltpu.sync_copy(src_h.at[wid], src_v)
        pltpu.sync_copy(dst_h.at[wid], dst_v)
        pltpu.sync_copy(w_h.at[wid], w_v)

        def _zero(i, _):
            r = i // DV
            t = i % DV
            fbuf[r, pl.ds(t * L, L)] = jnp.zeros((L,), jnp.float32)
            return 0

        lax.fori_loop(0, WB_CHUNK * DV, _zero, 0)

        for k in range(WB_STEPS):
            pltpu.sync_copy(fbuf, acc_s.at[pl.ds(s * ROWS_PT + k * WB_CHUNK, WB_CHUNK)])
        plsc.subcore_barrier()

        sct_v = src_v if scatter_by_src else dst_v

        def _scale(rv, j):
            def _grp(g, _):
                sv = src_v[j, pl.ds(g * L, L)]
                dv = dst_v[j, pl.ds(g * L, L)]
                wv = w_v[j, pl.ds(g * L, L)]
                wv = jnp.where(sv == dv, jnp.zeros((L,), jnp.float32), wv)
                for lane in range(L):
                    r = g * L + lane
                    wb = wv[lane]
                    for t in range(DV):
                        if p_is_ones:
                            rv[r, pl.ds(t * L, L)] = jnp.full((L,), wb)
                        elif p_bf16:
                            if t % 2 == 0:
                                vi = rv[r, pl.ds((t // 2) * L, L)]
                                ev = lax.bitcast_convert_type(vi << 16, jnp.float32)
                                od = lax.bitcast_convert_type(vi & jnp.int32(-65536), jnp.float32)
                                fbuf[r, pl.ds(t * L, L)] = ev * wb
                                fbuf[r, pl.ds((t + 1) * L, L)] = od * wb
                        else:
                            rv[r, pl.ds(t * L, L)] = rv[r, pl.ds(t * L, L)] * wb
                return 0

            lax.fori_loop(0, CH // L, _grp, 0)

        if p_is_ones:
            def _chunk(j, _):
                _scale(rows[0], j)
                pltpu.sync_copy(rows[0], acc_s.at[sct_v.at[j]], add=True)
                return 0

            lax.fori_loop(0, cc, _chunk, 0)
        else:
            for b in range(NBUF):
                pltpu.async_copy(p_h.at[src_v.at[b]], rows[b], gsem[b])

            def _round(jo, _):
                for b in range(NBUF):
                    j = jo * NBUF + b
                    pltpu.make_async_copy(p_h.at[src_v.at[j]], rows[b], gsem[b]).wait()
                    _scale(rows[b], j)
                    sctsrc = fbuf if p_bf16 else rows[b]
                    pltpu.sync_copy(sctsrc, acc_s.at[sct_v.at[j]], add=True)
                    pltpu.async_copy(p_h.at[src_v.at[j + NBUF]], rows[b], gsem[b])
                return 0

            lax.fori_loop(0, cc // NBUF - 1, _round, 0)
            for b in range(NBUF):
                j = cc - NBUF + b
                pltpu.make_async_copy(p_h.at[src_v.at[j]], rows[b], gsem[b]).wait()
                _scale(rows[b], j)
                sctsrc = fbuf if p_bf16 else rows[b]
                pltpu.sync_copy(sctsrc, acc_s.at[sct_v.at[j]], add=True)

        plsc.subcore_barrier()

        for k in range(WB_STEPS):
            base = s * ROWS_PT + k * WB_CHUNK
            wbuf = fbuf if p_bf16 else rows[k % 2]
            pltpu.sync_copy(acc_s.at[pl.ds(base, WB_CHUNK)], wbuf)
            pltpu.sync_copy(wbuf, out_h.at[c, pl.ds(base, WB_CHUNK)])

    return pl.kernel(
        body,
        out_type=jax.ShapeDtypeStruct((NC, N_PAD, D), jnp.float32),
        mesh=_MESH,
        scratch_types=[
            pltpu.VMEM((NCKM, CH), jnp.int32),
            pltpu.VMEM((NCKM, CH), jnp.int32),
            pltpu.VMEM((NCKM, CH), jnp.float32),
        ] + [pltpu.VMEM((CH, D // 2), jnp.int32) if p_bf16 else
             pltpu.VMEM((CH, D), jnp.float32)] * NBUF + (
            [pltpu.VMEM((CH, D), jnp.float32)] if p_bf16 else []) + [
            pltpu.VMEM_SHARED((N_PAD, D), jnp.float32),
        ] + [pltpu.SemaphoreType.DMA] * NBUF,
        compiler_params=pltpu.CompilerParams(use_tc_tiling_on_sc=False),
    )


def _wn_body(src_h, dst_h, ew_h, dinv_h, out_h, src_v, dst_v, ew_v, dinv_v, wn_v):
    c = lax.axis_index("c")
    s = lax.axis_index("s")
    wid = s * NC + c
    cc = jnp.where(c == FAST, NCKF, NCKS)
    pltpu.sync_copy(src_h.at[wid], src_v)
    pltpu.sync_copy(dst_h.at[wid], dst_v)
    pltpu.sync_copy(ew_h.at[wid], ew_v)
    pltpu.sync_copy(dinv_h, dinv_v)

    def _body(i, _):
        sv = src_v[pl.ds(i * L, L)]
        dv = dst_v[pl.ds(i * L, L)]
        wv = ew_v[pl.ds(i * L, L)]
        a = plsc.load_gather(dinv_v, [sv])
        b = plsc.load_gather(dinv_v, [dv])
        r = jnp.where(sv == dv, jnp.zeros((L,), jnp.float32), -(a * wv * b))
        wn_v[pl.ds(i * L, L)] = r
        return 0

    lax.fori_loop(0, cc * (CH // L), _body, 0)
    pltpu.sync_copy(wn_v, out_h.at[wid])


_wn_kernel = pl.kernel(
    _wn_body,
    out_type=jax.ShapeDtypeStruct((NW, EPT_MAX), jnp.float32),
    mesh=_MESH,
    scratch_types=[
        pltpu.VMEM((EPT_MAX,), jnp.int32),
        pltpu.VMEM((EPT_MAX,), jnp.int32),
        pltpu.VMEM((EPT_MAX,), jnp.float32),
        pltpu.VMEM((N,), jnp.float32),
        pltpu.VMEM((EPT_MAX,), jnp.float32),
    ],
    compiler_params=pltpu.CompilerParams(needs_layout_passes=False),
)


def _tc_prep_body(tdeg_ref, x_ref, w1_ref, dinv_ref, q1_ref):
    deg = tdeg_ref[0, :N, 0] + tdeg_ref[1, :N, 0]
    dinv = jnp.where(deg > 0, lax.rsqrt(jnp.maximum(deg, 1e-12)), 0.0)
    dinv_ref[...] = dinv
    q1_ref[...] = jnp.dot(x_ref[...], w1_ref[...], preferred_element_type=jnp.float32)


def _tc_prep(tdeg, x, w1p):
    return pl.pallas_call(
        _tc_prep_body,
        out_shape=[
            jax.ShapeDtypeStruct((N,), jnp.float32),
            jax.ShapeDtypeStruct((N, w1p.shape[1]), jnp.float32),
        ],
    )(tdeg, x, w1p)


def _bn_lrelu(pre, g, b, relu):
    m = jnp.mean(pre, axis=0)
    v = jnp.mean((pre - m) ** 2, axis=0)
    h = (pre - m) * lax.rsqrt(v + 1e-5) * g + b
    if relu:
        h = jnp.where(h > 0, h, 0.01 * h)
    return h


def _tc_layer_body(qmode, emit_bf16, h_ref, t_ref, w0_ref, w1_ref, b_ref, g_ref, bb_ref, *out_refs):
    ts = t_ref[0, :N] + t_ref[1, :N]
    pre = jnp.dot(h_ref[...], w0_ref[...], preferred_element_type=jnp.float32)
    if qmode:
        pre = pre + ts + b_ref[...]
    else:
        pre = pre + jnp.dot(ts, w1_ref[...], preferred_element_type=jnp.float32) + b_ref[...]
    h = _bn_lrelu(pre, g_ref[...], bb_ref[...], True)
    out_refs[0][...] = h


def _tc_layer(qmode, h, t, w0, w1, b, g, bb):
    return pl.pallas_call(
        functools.partial(_tc_layer_body, qmode, False),
        out_shape=[jax.ShapeDtypeStruct((N, w0.shape[1]), jnp.float32)],
    )(h, t, w0, w1, b, g, bb)[0]


def _tc_final_body(h_ref, t_ref, w0_ref, w1_ref, b_ref, g_ref, bb_ref, ids_ref,
                   f1w_ref, f1b_ref, f2w_ref, f2b_ref, f3w_ref, f3b_ref,
                   f4w_ref, f4b_ref, out_ref):
    ts = t_ref[0, :N] + t_ref[1, :N]
    pre = (jnp.dot(h_ref[...], w0_ref[...], preferred_element_type=jnp.float32)
           + jnp.dot(ts, w1_ref[...], preferred_element_type=jnp.float32)
           + b_ref[...])
    h4 = _bn_lrelu(pre, g_ref[...], bb_ref[...], False)
    ids = ids_ref[...]
    onehot = (lax.broadcasted_iota(jnp.int32, (NB, N), 0) == ids[None, :]).astype(jnp.float32)
    counts = jnp.sum(onehot, axis=1)
    pooled = jnp.dot(onehot, h4, preferred_element_type=jnp.float32)
    pooled = pooled / jnp.maximum(counts, 1.0)[:, None]
    gact = jnp.where(pooled > 0, pooled, 0.01 * pooled)
    z = jnp.maximum(jnp.dot(gact, f1w_ref[...], preferred_element_type=jnp.float32) + f1b_ref[...], 0.0)
    z = jnp.maximum(jnp.dot(z, f2w_ref[...], preferred_element_type=jnp.float32) + f2b_ref[...], 0.0)
    z = jnp.maximum(jnp.dot(z, f3w_ref[...], preferred_element_type=jnp.float32) + f3b_ref[...], 0.0)
    z = jnp.dot(z, f4w_ref[...], preferred_element_type=jnp.float32) + f4b_ref[...]
    out_ref[...] = 1.0 / (1.0 + jnp.exp(-z))


def _tc_final(h, t, w0, w1, b, g, bb, ids, f1w, f1b, f2w, f2b, f3w, f3b, f4w, f4b):
    return pl.pallas_call(
        _tc_final_body,
        out_shape=jax.ShapeDtypeStruct((NB, 1), jnp.float32),
    )(h, t, w0, w1, b, g, bb, ids, f1w, f1b, f2w, f2b, f3w, f3b, f4w, f4b)


_spmm_deg = _make_spmm(16, scatter_by_src=True, p_is_ones=True)
_spmm32 = _make_spmm(32)
_spmm128 = _make_spmm(128, p_bf16=True)


_BF16_PERM = np.concatenate([
    np.concatenate([np.arange(32 * t, 32 * t + 32, 2),
                    np.arange(32 * t + 1, 32 * t + 32, 2)])
    for t in range(4)])


def _pad_cols(a, n):
    return jnp.pad(a, ((0, 0), (0, n - a.shape[1])))


def _edge_layout(flat):
    nf = NS * NCKF * CH
    fast = flat[:nf].reshape(NS, NCKF, CH)
    slow = jnp.pad(flat[nf:].reshape(NS, NCKS, CH),
                   ((0, 0), (0, NCKM - NCKS), (0, 0)))
    pair = (fast, slow) if FAST == 0 else (slow, fast)
    return jnp.stack(pair, axis=1).reshape(NW, NCKM, CH)


def kernel(x, edge_index, edge_weight, batch_ids,
           c1w0, c1w1, c1b, bn1g, bn1b,
           c2w0, c2w1, c2b, bn2g, bn2b,
           c3w0, c3w1, c3b, bn3g, bn3b,
           c4w0, c4w1, c4b, bn4g, bn4b,
           f1w, f1b, f2w, f2b, f3w, f3b, f4w, f4b):
    E = edge_weight.shape[0]
    pad = E_PAD - E
    src3 = _edge_layout(jnp.concatenate([edge_index[0], jnp.zeros((pad,), jnp.int32)]))
    dst3 = _edge_layout(jnp.concatenate([edge_index[1], jnp.zeros((pad,), jnp.int32)]))
    ew3 = _edge_layout(jnp.concatenate([edge_weight, jnp.zeros((pad,), jnp.float32)]))

    c1w0p = _pad_cols(c1w0, 32)
    c1w1p = _pad_cols(c1w1, 32)
    c1bp = jnp.pad(c1b, (0, 2))
    bn1gp = jnp.pad(bn1g, (0, 2))
    bn1bp = jnp.pad(bn1b, (0, 2))
    c2w0p = jnp.pad(c2w0, ((0, 2), (0, 0)))
    c2w1p = jnp.pad(c2w1, ((0, 2), (0, 0)))

    ones16 = jnp.zeros((N, 16), jnp.float32)
    tdeg = _spmm_deg(ones16, src3, dst3, ew3)
    dinv, q1 = _tc_prep(tdeg, x, c1w1p)
    wn3 = _wn_kernel(src3.reshape(NW, EPT_MAX), dst3.reshape(NW, EPT_MAX),
                     ew3.reshape(NW, EPT_MAX), dinv).reshape(NW, NCKM, CH)

    perm = jnp.array(_BF16_PERM)
    c3w1p = c3w1[perm]
    c4w1p = c4w1[perm]

    def _pack_bf16(h):
        hb = h.astype(jnp.bfloat16)
        return jax.lax.bitcast_convert_type(
            hb.reshape(N, h.shape[1] // 2, 2), jnp.int32)

    t1 = _spmm32(q1, src3, dst3, wn3)
    h1 = _tc_layer(True, x, t1, c1w0p, c1w0p, c1bp, bn1gp, bn1bp)
    t2 = _spmm32(h1, src3, dst3, wn3)
    h2 = _tc_layer(False, h1, t2, c2w0p, c2w1p, c2b, bn2g, bn2b)
    t3 = _spmm128(_pack_bf16(h2), src3, dst3, wn3)
    h3 = _tc_layer(False, h2, t3, c3w0, c3w1p, c3b, bn3g, bn3b)
    t4 = _spmm128(_pack_bf16(h3), src3, dst3, wn3)
    return _tc_final(h3, t4, c4w0, c4w1p, c4b, bn4g, bn4b, batch_ids,
                     f1w, f1b, f2w, f2b, f3w, f3b, f4w, f4b)


# speedup vs baseline: 1.2265x; 1.2265x over previous
import functools

import jax
import jax.numpy as jnp
from jax import lax
from jax.experimental import pallas as pl
from jax.experimental.pallas import tpu as pltpu
from jax.experimental.pallas import tpu_sc as plsc

N = 10000
F_IN = 128
NB = 16

NC = 2
NS = 16
L = 16
NW = NC * NS

CH = 64
FAST = 0
NCKF = 132
NCKS = 28
NCKM = NCKF
EPT_MAX = NCKM * CH
E_PAD = NS * (NCKF + NCKS) * CH

N_PAD = 10240
ROWS_PT = N_PAD // NS
WB_CHUNK = 64
WB_STEPS = ROWS_PT // WB_CHUNK

_MESH = plsc.VectorSubcoreMesh(core_axis_name="c", subcore_axis_name="s")


def _make_spmm(D, scatter_by_src=False, p_is_ones=False):
    DV = D // L
    NBUF = 2 if D > 32 else 4

    def body(p_h, src_h, dst_h, w_h, out_h, src_v, dst_v, w_v, *rest):
        rows = rest[:NBUF]
        acc_s = rest[NBUF]
        gsem = rest[NBUF + 1:]
        c = lax.axis_index("c")
        s = lax.axis_index("s")
        wid = s * NC + c
        cc = jnp.where(c == FAST, NCKF, NCKS)

        pltpu.sync_copy(src_h.at[wid], src_v)
        pltpu.sync_copy(dst_h.at[wid], dst_v)
        pltpu.sync_copy(w_h.at[wid], w_v)

        def _zero(i, _):
            r = i // DV
            t = i % DV
            rows[0][r, pl.ds(t * L, L)] = jnp.zeros((L,), jnp.float32)
            return 0

        lax.fori_loop(0, WB_CHUNK * DV, _zero, 0)

        for k in range(WB_STEPS):
            pltpu.sync_copy(rows[0], acc_s.at[pl.ds(s * ROWS_PT + k * WB_CHUNK, WB_CHUNK)])
        plsc.subcore_barrier()

        sct_v = src_v if scatter_by_src else dst_v

        def _scale(rv, j):
            @plsc.parallel_loop(0, CH // L, unroll=2)
            def _grp(g):
                sv = src_v[j, pl.ds(g * L, L)]
                dv = dst_v[j, pl.ds(g * L, L)]
                wv = w_v[j, pl.ds(g * L, L)]
                wv = jnp.where(sv == dv, jnp.zeros((L,), jnp.float32), wv)
                for lane in range(L):
                    r = g * L + lane
                    wb = wv[lane]
                    for t in range(DV):
                        if p_is_ones:
                            rv[r, pl.ds(t * L, L)] = jnp.full((L,), wb)
                        else:
                            rv[r, pl.ds(t * L, L)] = rv[r, pl.ds(t * L, L)] * wb

        if p_is_ones:
            def _chunk(j, _):
                _scale(rows[0], j)
                pltpu.sync_copy(rows[0], acc_s.at[sct_v.at[j]], add=True)
                return 0

            lax.fori_loop(0, cc, _chunk, 0)
        else:
            for b in range(NBUF):
                pltpu.async_copy(p_h.at[src_v.at[b]], rows[b], gsem[b])

            def _round(jo, _):
                for b in range(NBUF):
                    j = jo * NBUF + b
                    pltpu.make_async_copy(p_h.at[src_v.at[j]], rows[b], gsem[b]).wait()
                    _scale(rows[b], j)
                    pltpu.sync_copy(rows[b], acc_s.at[sct_v.at[j]], add=True)
                    pltpu.async_copy(p_h.at[src_v.at[j + NBUF]], rows[b], gsem[b])
                return 0

            lax.fori_loop(0, cc // NBUF - 1, _round, 0)
            for b in range(NBUF):
                j = cc - NBUF + b
                pltpu.make_async_copy(p_h.at[src_v.at[j]], rows[b], gsem[b]).wait()
                _scale(rows[b], j)
                pltpu.sync_copy(rows[b], acc_s.at[sct_v.at[j]], add=True)

        plsc.subcore_barrier()

        for k in range(WB_STEPS):
            base = s * ROWS_PT + k * WB_CHUNK
            pltpu.sync_copy(acc_s.at[pl.ds(base, WB_CHUNK)], rows[k % 2])
            pltpu.sync_copy(rows[k % 2], out_h.at[c, pl.ds(base, WB_CHUNK)])

    return pl.kernel(
        body,
        out_type=jax.ShapeDtypeStruct((NC, N_PAD, D), jnp.float32),
        mesh=_MESH,
        scratch_types=[
            pltpu.VMEM((NCKM, CH), jnp.int32),
            pltpu.VMEM((NCKM, CH), jnp.int32),
            pltpu.VMEM((NCKM, CH), jnp.float32),
        ] + [pltpu.VMEM((CH, D), jnp.float32)] * NBUF + [
            pltpu.VMEM_SHARED((N_PAD, D), jnp.float32),
        ] + [pltpu.SemaphoreType.DMA] * NBUF,
        compiler_params=pltpu.CompilerParams(use_tc_tiling_on_sc=False),
    )


def _wn_body(src_h, dst_h, ew_h, dinv_h, out_h, src_v, dst_v, ew_v, dinv_v, wn_v):
    c = lax.axis_index("c")
    s = lax.axis_index("s")
    wid = s * NC + c
    cc = jnp.where(c == FAST, NCKF, NCKS)
    pltpu.sync_copy(src_h.at[wid], src_v)
    pltpu.sync_copy(dst_h.at[wid], dst_v)
    pltpu.sync_copy(ew_h.at[wid], ew_v)
    pltpu.sync_copy(dinv_h, dinv_v)

    def _body(i, _):
        sv = src_v[pl.ds(i * L, L)]
        dv = dst_v[pl.ds(i * L, L)]
        wv = ew_v[pl.ds(i * L, L)]
        a = plsc.load_gather(dinv_v, [sv])
        b = plsc.load_gather(dinv_v, [dv])
        r = jnp.where(sv == dv, jnp.zeros((L,), jnp.float32), -(a * wv * b))
        wn_v[pl.ds(i * L, L)] = r
        return 0

    lax.fori_loop(0, cc * (CH // L), _body, 0)
    pltpu.sync_copy(wn_v, out_h.at[wid])


_wn_kernel = pl.kernel(
    _wn_body,
    out_type=jax.ShapeDtypeStruct((NW, EPT_MAX), jnp.float32),
    mesh=_MESH,
    scratch_types=[
        pltpu.VMEM((EPT_MAX,), jnp.int32),
        pltpu.VMEM((EPT_MAX,), jnp.int32),
        pltpu.VMEM((EPT_MAX,), jnp.float32),
        pltpu.VMEM((N,), jnp.float32),
        pltpu.VMEM((EPT_MAX,), jnp.float32),
    ],
    compiler_params=pltpu.CompilerParams(needs_layout_passes=False),
)


def _tc_prep_body(tdeg_ref, x_ref, w1_ref, dinv_ref, q1_ref):
    deg = tdeg_ref[0, :N, 0] + tdeg_ref[1, :N, 0]
    dinv = jnp.where(deg > 0, lax.rsqrt(jnp.maximum(deg, 1e-12)), 0.0)
    dinv_ref[...] = dinv
    q1_ref[...] = jnp.dot(x_ref[...], w1_ref[...], preferred_element_type=jnp.float32)


def _tc_prep(tdeg, x, w1p):
    return pl.pallas_call(
        _tc_prep_body,
        out_shape=[
            jax.ShapeDtypeStruct((N,), jnp.float32),
            jax.ShapeDtypeStruct((N, w1p.shape[1]), jnp.float32),
        ],
    )(tdeg, x, w1p)


def _bn_lrelu(pre, g, b, relu):
    m = jnp.mean(pre, axis=0)
    v = jnp.mean((pre - m) ** 2, axis=0)
    h = (pre - m) * lax.rsqrt(v + 1e-5) * g + b
    if relu:
        h = jnp.where(h > 0, h, 0.01 * h)
    return h


def _tc_layer_body(qmode, h_ref, t_ref, w0_ref, w1_ref, b_ref, g_ref, bb_ref, out_ref):
    ts = t_ref[0, :N] + t_ref[1, :N]
    pre = jnp.dot(h_ref[...], w0_ref[...], preferred_element_type=jnp.float32)
    if qmode:
        pre = pre + ts + b_ref[...]
    else:
        pre = pre + jnp.dot(ts, w1_ref[...], preferred_element_type=jnp.float32) + b_ref[...]
    out_ref[...] = _bn_lrelu(pre, g_ref[...], bb_ref[...], True)


def _tc_layer(qmode, h, t, w0, w1, b, g, bb):
    return pl.pallas_call(
        functools.partial(_tc_layer_body, qmode),
        out_shape=jax.ShapeDtypeStruct((N, w0.shape[1]), jnp.float32),
    )(h, t, w0, w1, b, g, bb)


def _tc_final_body(h_ref, t_ref, w0_ref, w1_ref, b_ref, g_ref, bb_ref, ids_ref,
                   f1w_ref, f1b_ref, f2w_ref, f2b_ref, f3w_ref, f3b_ref,
                   f4w_ref, f4b_ref, out_ref):
    ts = t_ref[0, :N] + t_ref[1, :N]
    pre = (jnp.dot(h_ref[...], w0_ref[...], preferred_element_type=jnp.float32)
           + jnp.dot(ts, w1_ref[...], preferred_element_type=jnp.float32)
           + b_ref[...])
    h4 = _bn_lrelu(pre, g_ref[...], bb_ref[...], False)
    ids = ids_ref[...]
    onehot = (lax.broadcasted_iota(jnp.int32, (NB, N), 0) == ids[None, :]).astype(jnp.float32)
    counts = jnp.sum(onehot, axis=1)
    pooled = jnp.dot(onehot, h4, preferred_element_type=jnp.float32)
    pooled = pooled / jnp.maximum(counts, 1.0)[:, None]
    gact = jnp.where(pooled > 0, pooled, 0.01 * pooled)
    z = jnp.maximum(jnp.dot(gact, f1w_ref[...], preferred_element_type=jnp.float32) + f1b_ref[...], 0.0)
    z = jnp.maximum(jnp.dot(z, f2w_ref[...], preferred_element_type=jnp.float32) + f2b_ref[...], 0.0)
    z = jnp.maximum(jnp.dot(z, f3w_ref[...], preferred_element_type=jnp.float32) + f3b_ref[...], 0.0)
    z = jnp.dot(z, f4w_ref[...], preferred_element_type=jnp.float32) + f4b_ref[...]
    out_ref[...] = 1.0 / (1.0 + jnp.exp(-z))


def _tc_final(h, t, w0, w1, b, g, bb, ids, f1w, f1b, f2w, f2b, f3w, f3b, f4w, f4b):
    return pl.pallas_call(
        _tc_final_body,
        out_shape=jax.ShapeDtypeStruct((NB, 1), jnp.float32),
    )(h, t, w0, w1, b, g, bb, ids, f1w, f1b, f2w, f2b, f3w, f3b, f4w, f4b)


_spmm_deg = _make_spmm(16, scatter_by_src=True, p_is_ones=True)
_spmm32 = _make_spmm(32)
_spmm128 = _make_spmm(128)


def _pad_cols(a, n):
    return jnp.pad(a, ((0, 0), (0, n - a.shape[1])))


def _edge_layout(flat):
    nf = NS * NCKF * CH
    fast = flat[:nf].reshape(NS, NCKF, CH)
    slow = jnp.pad(flat[nf:].reshape(NS, NCKS, CH),
                   ((0, 0), (0, NCKM - NCKS), (0, 0)))
    pair = (fast, slow) if FAST == 0 else (slow, fast)
    return jnp.stack(pair, axis=1).reshape(NW, NCKM, CH)


def kernel(x, edge_index, edge_weight, batch_ids,
           c1w0, c1w1, c1b, bn1g, bn1b,
           c2w0, c2w1, c2b, bn2g, bn2b,
           c3w0, c3w1, c3b, bn3g, bn3b,
           c4w0, c4w1, c4b, bn4g, bn4b,
           f1w, f1b, f2w, f2b, f3w, f3b, f4w, f4b):
    E = edge_weight.shape[0]
    pad = E_PAD - E
    src3 = _edge_layout(jnp.concatenate([edge_index[0], jnp.zeros((pad,), jnp.int32)]))
    dst3 = _edge_layout(jnp.concatenate([edge_index[1], jnp.zeros((pad,), jnp.int32)]))
    ew3 = _edge_layout(jnp.concatenate([edge_weight, jnp.zeros((pad,), jnp.float32)]))

    c1w0p = _pad_cols(c1w0, 32)
    c1w1p = _pad_cols(c1w1, 32)
    c1bp = jnp.pad(c1b, (0, 2))
    bn1gp = jnp.pad(bn1g, (0, 2))
    bn1bp = jnp.pad(bn1b, (0, 2))
    c2w0p = jnp.pad(c2w0, ((0, 2), (0, 0)))
    c2w1p = jnp.pad(c2w1, ((0, 2), (0, 0)))

    ones16 = jnp.zeros((N, 16), jnp.float32)
    tdeg = _spmm_deg(ones16, src3, dst3, ew3)
    dinv, q1 = _tc_prep(tdeg, x, c1w1p)
    wn3 = _wn_kernel(src3.reshape(NW, EPT_MAX), dst3.reshape(NW, EPT_MAX),
                     ew3.reshape(NW, EPT_MAX), dinv).reshape(NW, NCKM, CH)

    t1 = _spmm32(q1, src3, dst3, wn3)
    h1 = _tc_layer(True, x, t1, c1w0p, c1w0p, c1bp, bn1gp, bn1bp)
    t2 = _spmm32(h1, src3, dst3, wn3)
    h2 = _tc_layer(False, h1, t2, c2w0p, c2w1p, c2b, bn2g, bn2b)
    t3 = _spmm128(h2, src3, dst3, wn3)
    h3 = _tc_layer(False, h2, t3, c3w0, c3w1, c3b, bn3g, bn3b)
    t4 = _spmm128(h3, src3, dst3, wn3)
    return _tc_final(h3, t4, c4w0, c4w1, c4b, bn4g, bn4b, batch_ids,
                     f1w, f1b, f2w, f2b, f3w, f3b, f4w, f4b)
